# baseline (speedup 1.0000x reference)
import functools

import jax
import jax.numpy as jnp
import numpy as np
from jax import lax
from jax.experimental import pallas as pl
from jax.experimental.pallas import tpu as pltpu
from jax.experimental.pallas import tpu_sc as plsc

D_MODEL = 512
NHEAD = 8
DH = D_MODEL // NHEAD
KSEL = 16
CHUNK = 128
TILE = 2048
NEG = float(np.float32(-3e38))

SC_NC = 2
SC_NS = 16
SC_NW = SC_NC * SC_NS


def _sims_body(b, n, zq_ref, keys_ref, knorm_ref, sims_ref, cm_ref):
    ti = pl.program_id(0)
    cpt = TILE // CHUNK
    qn = zq_ref[...]
    kn = keys_ref[...] / knorm_ref[...]
    sims = lax.dot_general(qn, kn, (((1,), (1,)), ((), ())),
                           preferred_element_type=jnp.float32)
    col = ti * TILE + lax.broadcasted_iota(jnp.int32, (b, TILE), 1)
    sims = jnp.where(col < n, sims, NEG)
    sims_ref[...] = sims
    cm_ref[0] = jnp.max(sims.reshape(b, cpt, CHUNK), axis=2)


def _sims_call(qn, keys, knorm_pad, npad, n):
    b = qn.shape[0]
    nt = npad // TILE
    cpt = TILE // CHUNK
    body = functools.partial(_sims_body, b, n)
    return pl.pallas_call(
        body,
        grid=(nt,),
        in_specs=[
            pl.BlockSpec((b, D_MODEL), lambda i: (0, 0)),
            pl.BlockSpec((TILE, D_MODEL), lambda i: (i, 0)),
            pl.BlockSpec((TILE, 1), lambda i: (i, 0)),
        ],
        out_specs=[
            pl.BlockSpec((b, TILE), lambda i: (0, i)),
            pl.BlockSpec((1, b, cpt), lambda i: (i, 0, 0)),
        ],
        out_shape=[
            jax.ShapeDtypeStruct((b, npad), jnp.float32),
            jax.ShapeDtypeStruct((nt, b, cpt), jnp.float32),
        ],
    )(qn, keys, knorm_pad)


def _chunksel_body(b, nchunk, cm_ref, rowid_ref):
    work = cm_ref[...]
    gid = lax.broadcasted_iota(jnp.int32, (b, nchunk), 1)
    big = jnp.int32(2 ** 30)
    cols = []
    for _ in range(KSEL):
        m = jnp.max(work, axis=1)
        p = jnp.min(jnp.where(work == m[:, None], gid, big), axis=1)
        cols.append(p.reshape(b, 1))
        work = jnp.where(gid == p[:, None], NEG, work)
    brow = lax.broadcasted_iota(jnp.int32, (b, 1), 0) * nchunk
    rowid_ref[...] = jnp.concatenate(cols, axis=1) + brow


def _chunksel_call(cm):
    b, nchunk = cm.shape
    body = functools.partial(_chunksel_body, b, nchunk)
    return pl.pallas_call(
        body,
        out_shape=jax.ShapeDtypeStruct((b, KSEL), jnp.int32),
    )(cm)


def _sc_gather(table, idx, d):
    rows = idx.shape[0]
    per_w = rows // SC_NW
    j_steps = per_w // CHUNK
    idx3 = idx.reshape(SC_NW, j_steps, CHUNK)
    mesh = plsc.VectorSubcoreMesh(core_axis_name="c", subcore_axis_name="s")

    @functools.partial(
        pl.kernel,
        mesh=mesh,
        out_type=jax.ShapeDtypeStruct((rows, d), jnp.float32),
        scratch_types=[
            pltpu.VMEM((j_steps, CHUNK), jnp.int32),
            pltpu.VMEM((CHUNK, d), jnp.float32),
            pltpu.SemaphoreType.DMA,
        ],
    )
    def k(table_hbm, idx_hbm, out_hbm, idx_v, rows_v, sem):
        wid = lax.axis_index("s") * SC_NC + lax.axis_index("c")
        pltpu.sync_copy(idx_hbm.at[wid], idx_v)
        for j in range(j_steps):
            pltpu.async_copy(table_hbm.at[idx_v.at[j]], rows_v, sem).wait()
            pltpu.sync_copy(rows_v, out_hbm.at[pl.ds(wid * per_w + j * CHUNK, CHUNK)])

    return k(table, idx3)


def _select_body(b, nchunk, cand_ref, rowid_ref, tv_ref, ti_ref):
    bi = pl.program_id(0)
    cand = cand_ref[...]
    rowid = rowid_ref[...]
    brow = (bi * b + lax.broadcasted_iota(jnp.int32, (b, KSEL), 0)) * nchunk
    base = (rowid - brow) * CHUNK
    lane = lax.broadcasted_iota(jnp.int32, (b, KSEL, CHUNK), 2)
    gidx = (base.reshape(b, KSEL, 1) + lane).reshape(b, KSEL * CHUNK)
    big = jnp.int32(2 ** 30)
    vals, idxs = [], []
    for _ in range(KSEL):
        m = jnp.max(cand, axis=1)
        p = jnp.min(jnp.where(cand == m[:, None], gidx, big), axis=1)
        vals.append(m.reshape(b, 1))
        idxs.append(p.reshape(b, 1))
        cand = jnp.where(gidx == p[:, None], NEG, cand)
    tv_ref[...] = jnp.concatenate(vals, axis=1)
    ti_ref[...] = jnp.concatenate(idxs, axis=1)


def _select_call(cand, rowid, nchunk):
    b = cand.shape[0]
    bq = 256 if b % 256 == 0 else b
    nblk = b // bq
    body = functools.partial(_select_body, bq, nchunk)
    return pl.pallas_call(
        body,
        grid=(nblk,),
        in_specs=[
            pl.BlockSpec((bq, KSEL * CHUNK), lambda i: (i, 0)),
            pl.BlockSpec((bq, KSEL), lambda i: (i, 0)),
        ],
        out_specs=[
            pl.BlockSpec((bq, KSEL), lambda i: (i, 0)),
            pl.BlockSpec((bq, KSEL), lambda i: (i, 0)),
        ],
        out_shape=[
            jax.ShapeDtypeStruct((b, KSEL), jnp.float32),
            jax.ShapeDtypeStruct((b, KSEL), jnp.int32),
        ],
    )(cand, rowid)


def _fusion_body(bq, zq_ref, ctx_ref, wq_ref, bq_ref, wkp_ref, bkp_ref, wv_ref,
                 bv_ref, wo_ref, bo_ref, lng_ref, lnb_ref, wf1_ref, bf1_ref,
                 wf2_ref, bf2_ref, wa1_ref, ba1_ref, wa2_ref, ba2_ref,
                 fc_ref, an_ref):
    zq = zq_ref[...]
    ctx = ctx_ref[...]
    cn = ctx / (jnp.sqrt(jnp.sum(ctx * ctx, axis=1, keepdims=True)) + 1e-8)
    q = jnp.dot(zq, wq_ref[...], preferred_element_type=jnp.float32) + bq_ref[...]
    kk = jnp.dot(cn, wkp_ref[...], preferred_element_type=jnp.float32) + bkp_ref[...]
    vv = jnp.dot(cn, wv_ref[...], preferred_element_type=jnp.float32) + bv_ref[...]
    scale = jnp.float32(1.0 / (DH ** 0.5))
    sel = (lax.broadcasted_iota(jnp.int32, (D_MODEL, NHEAD), 0) // DH
           == lax.broadcasted_iota(jnp.int32, (D_MODEL, NHEAD), 1)
           ).astype(jnp.float32)
    selt = (lax.broadcasted_iota(jnp.int32, (NHEAD, D_MODEL), 1) // DH
            == lax.broadcasted_iota(jnp.int32, (NHEAD, D_MODEL), 0)
            ).astype(jnp.float32)
    qe = jnp.broadcast_to(q.reshape(bq, 1, D_MODEL),
                          (bq, KSEL, D_MODEL)).reshape(bq * KSEL, D_MODEL)
    p = qe * kk
    l3 = (jnp.dot(p, sel, precision=lax.Precision.HIGHEST,
                  preferred_element_type=jnp.float32)
          * scale).reshape(bq, KSEL, NHEAD)
    l3 = l3 - jnp.max(l3, axis=1, keepdims=True)
    e3 = jnp.exp(l3)
    a3 = e3 / jnp.sum(e3, axis=1, keepdims=True)
    ae = jnp.dot(a3.reshape(bq * KSEL, NHEAD), selt,
                 precision=lax.Precision.HIGHEST,
                 preferred_element_type=jnp.float32)
    o = jnp.sum((ae * vv).reshape(bq, KSEL, D_MODEL), axis=1)
    c_attn = jnp.dot(o, wo_ref[...], preferred_element_type=jnp.float32) + bo_ref[...]
    x = c_attn + zq
    mu = jnp.mean(x, axis=1, keepdims=True)
    xm = x - mu
    var = jnp.mean(xm * xm, axis=1, keepdims=True)
    zf = xm / jnp.sqrt(var + 1e-5) * lng_ref[...] + lnb_ref[...]
    fused = jnp.concatenate([zq, zf], axis=1)
    f1 = jax.nn.gelu(jnp.dot(fused, wf1_ref[...], preferred_element_type=jnp.float32)
                     + bf1_ref[...])
    fc_ref[...] = jnp.dot(f1, wf2_ref[...], preferred_element_type=jnp.float32) + bf2_ref[...]
    a1 = jax.nn.gelu(jnp.dot(fused, wa1_ref[...], preferred_element_type=jnp.float32)
                     + ba1_ref[...])
    a2 = jnp.sum(a1 * wa2_ref[...], axis=1, keepdims=True) + ba2_ref[...]
    an_ref[...] = 1.0 / (1.0 + jnp.exp(-a2))


def _fusion_call(z_q, ctx, wqt, bq1, wkpt, bkp1, wvt, bv1, wot, bo1, lng1, lnb1,
                 wf1t, bf11, wf2t, bf21, wa1t, ba11, wa2, ba21, horizon):
    b = z_q.shape[0]
    bq = 128 if b % 128 == 0 else b
    nblk = b // bq
    body = functools.partial(_fusion_body, bq)
    const = lambda shape: pl.BlockSpec(shape, lambda i: (0, 0))
    return pl.pallas_call(
        body,
        grid=(nblk,),
        in_specs=[
            pl.BlockSpec((bq, D_MODEL), lambda i: (i, 0)),
            pl.BlockSpec((bq * KSEL, D_MODEL), lambda i: (i, 0)),
            const((D_MODEL, D_MODEL)), const((1, D_MODEL)),
            const((D_MODEL, D_MODEL)), const((1, D_MODEL)),
            const((D_MODEL, D_MODEL)), const((1, D_MODEL)),
            const((D_MODEL, D_MODEL)), const((1, D_MODEL)),
            const((1, D_MODEL)), const((1, D_MODEL)),
            const((2 * D_MODEL, D_MODEL)), const((1, D_MODEL)),
            const((D_MODEL, horizon)), const((1, horizon)),
            const((2 * D_MODEL, 64)), const((1, 64)),
            const((1, 64)), const((1, 1)),
        ],
        out_specs=[
            pl.BlockSpec((bq, horizon), lambda i: (i, 0)),
            pl.BlockSpec((bq, 1), lambda i: (i, 0)),
        ],
        out_shape=[
            jax.ShapeDtypeStruct((b, horizon), jnp.float32),
            jax.ShapeDtypeStruct((b, 1), jnp.float32),
        ],
    )(z_q, ctx, wqt, bq1, wkpt, bkp1, wvt, bv1, wot, bo1, lng1, lnb1,
      wf1t, bf11, wf2t, bf21, wa1t, ba11, wa2, ba21)


def kernel(z_q, keys, k, Wq, bq, Wkp, bkp, Wv, bv, Wo, bo, ln_g, ln_b,
           Wf1, bf1, Wf2, bf2, Wa1, ba1, Wa2, ba2):
    del k
    b, d = z_q.shape
    n = keys.shape[0]
    npad = ((n + TILE - 1) // TILE) * TILE
    nchunk = npad // CHUNK
    horizon = Wf2.shape[0]
    qn = z_q / (jnp.linalg.norm(z_q, axis=-1, keepdims=True) + 1e-8)
    knorm = jnp.linalg.norm(keys, axis=-1, keepdims=True) + 1e-8
    knorm_pad = jnp.pad(knorm, ((0, npad - n), (0, 0)), constant_values=1.0)

    sims, cm3 = _sims_call(qn, keys, knorm_pad, npad, n)
    cm = cm3.transpose(1, 0, 2).reshape(b, nchunk)
    rowid = _chunksel_call(cm)
    cand = _sc_gather(sims.reshape(b * nchunk, CHUNK), rowid.reshape(-1), CHUNK)
    top_vals, top_idx = _select_call(cand.reshape(b, KSEL * CHUNK), rowid, nchunk)
    ctx = _sc_gather(keys, top_idx.reshape(-1), d)

    forecast, anomaly = _fusion_call(
        z_q, ctx, Wq.T, bq.reshape(1, -1), Wkp.T, bkp.reshape(1, -1),
        Wv.T, bv.reshape(1, -1), Wo.T, bo.reshape(1, -1),
        ln_g.reshape(1, -1), ln_b.reshape(1, -1), Wf1.T, bf1.reshape(1, -1),
        Wf2.T, bf2.reshape(1, -1), Wa1.T, ba1.reshape(1, -1),
        Wa2, ba2.reshape(1, 1), horizon)
    return forecast, anomaly, top_vals, top_idx


# speedup vs baseline: 1.0190x; 1.0190x over previous
import functools

import jax
import jax.numpy as jnp
import numpy as np
from jax import lax
from jax.experimental import pallas as pl
from jax.experimental.pallas import tpu as pltpu
from jax.experimental.pallas import tpu_sc as plsc

D_MODEL = 512
NHEAD = 8
DH = D_MODEL // NHEAD
KSEL = 16
CHUNK = 128
TILE = 2048
NEG = float(np.float32(-3e38))

SC_NC = 2
SC_NS = 16
SC_NW = SC_NC * SC_NS


def _sims_body(b, n, zq_ref, keys_ref, knorm_ref, sims_ref, cm_ref):
    ti = pl.program_id(0)
    cpt = TILE // CHUNK
    qn = zq_ref[...]
    kn = keys_ref[...] / knorm_ref[...]
    sims = lax.dot_general(qn, kn, (((1,), (1,)), ((), ())),
                           preferred_element_type=jnp.float32)
    col = ti * TILE + lax.broadcasted_iota(jnp.int32, (b, TILE), 1)
    sims = jnp.where(col < n, sims, NEG)
    sims_ref[...] = sims
    cm_ref[0] = jnp.max(sims.reshape(b, cpt, CHUNK), axis=2)


def _sims_call(qn, keys, knorm_pad, npad, n):
    b = qn.shape[0]
    nt = npad // TILE
    cpt = TILE // CHUNK
    body = functools.partial(_sims_body, b, n)
    return pl.pallas_call(
        body,
        grid=(nt,),
        in_specs=[
            pl.BlockSpec((b, D_MODEL), lambda i: (0, 0)),
            pl.BlockSpec((TILE, D_MODEL), lambda i: (i, 0)),
            pl.BlockSpec((TILE, 1), lambda i: (i, 0)),
        ],
        out_specs=[
            pl.BlockSpec((b, TILE), lambda i: (0, i)),
            pl.BlockSpec((1, b, cpt), lambda i: (i, 0, 0)),
        ],
        out_shape=[
            jax.ShapeDtypeStruct((b, npad), jnp.float32),
            jax.ShapeDtypeStruct((nt, b, cpt), jnp.float32),
        ],
    )(qn, keys, knorm_pad)


def _chunksel_body(b, nchunk, cm_ref, rowid_ref):
    work = cm_ref[...]
    gid = lax.broadcasted_iota(jnp.int32, (b, nchunk), 1)
    big = jnp.int32(2 ** 30)
    cols = []
    for _ in range(KSEL):
        m = jnp.max(work, axis=1)
        p = jnp.min(jnp.where(work == m[:, None], gid, big), axis=1)
        cols.append(p.reshape(b, 1))
        work = jnp.where(gid == p[:, None], NEG, work)
    brow = lax.broadcasted_iota(jnp.int32, (b, 1), 0) * nchunk
    rowid_ref[...] = jnp.concatenate(cols, axis=1) + brow


def _chunksel_call(cm):
    b, nchunk = cm.shape
    body = functools.partial(_chunksel_body, b, nchunk)
    return pl.pallas_call(
        body,
        out_shape=jax.ShapeDtypeStruct((b, KSEL), jnp.int32),
    )(cm)


def _sc_gather(table, idx, d):
    rows = idx.shape[0]
    per_w = rows // SC_NW
    rp = 128 if d <= 128 else 64
    j_steps = per_w // rp
    idx3 = idx.reshape(SC_NW, j_steps, rp)
    mesh = plsc.VectorSubcoreMesh(core_axis_name="c", subcore_axis_name="s")

    @functools.partial(
        pl.kernel,
        mesh=mesh,
        out_type=jax.ShapeDtypeStruct((rows, d), jnp.float32),
        scratch_types=[
            pltpu.VMEM((j_steps, rp), jnp.int32),
            pltpu.VMEM((2, rp, d), jnp.float32),
            pltpu.SemaphoreType.DMA,
        ],
    )
    def k(table_hbm, idx_hbm, out_hbm, idx_v, rows_v, sem):
        wid = lax.axis_index("s") * SC_NC + lax.axis_index("c")
        base = wid * per_w
        pltpu.sync_copy(idx_hbm.at[wid], idx_v)
        cps = []
        for j in range(j_steps):
            cps.append(pltpu.async_copy(
                table_hbm.at[idx_v.at[j]], rows_v.at[j % 2], sem))
            if j >= 1:
                cps[j - 1].wait()
                pltpu.sync_copy(rows_v.at[(j - 1) % 2],
                                out_hbm.at[pl.ds(base + (j - 1) * rp, rp)])
        cps[j_steps - 1].wait()
        pltpu.sync_copy(rows_v.at[(j_steps - 1) % 2],
                        out_hbm.at[pl.ds(base + (j_steps - 1) * rp, rp)])

    return k(table, idx3)


def _select_body(b, nchunk, cand_ref, rowid_ref, tv_ref, ti_ref):
    bi = pl.program_id(0)
    cand = cand_ref[...]
    rowid = rowid_ref[...]
    brow = (bi * b + lax.broadcasted_iota(jnp.int32, (b, KSEL), 0)) * nchunk
    base = (rowid - brow) * CHUNK
    lane = lax.broadcasted_iota(jnp.int32, (b, KSEL, CHUNK), 2)
    gidx = (base.reshape(b, KSEL, 1) + lane).reshape(b, KSEL * CHUNK)
    big = jnp.int32(2 ** 30)
    vals, idxs = [], []
    for _ in range(KSEL):
        m = jnp.max(cand, axis=1)
        p = jnp.min(jnp.where(cand == m[:, None], gidx, big), axis=1)
        vals.append(m.reshape(b, 1))
        idxs.append(p.reshape(b, 1))
        cand = jnp.where(gidx == p[:, None], NEG, cand)
    tv_ref[...] = jnp.concatenate(vals, axis=1)
    ti_ref[...] = jnp.concatenate(idxs, axis=1)


def _select_call(cand, rowid, nchunk):
    b = cand.shape[0]
    bq = 256 if b % 256 == 0 else b
    nblk = b // bq
    body = functools.partial(_select_body, bq, nchunk)
    return pl.pallas_call(
        body,
        grid=(nblk,),
        in_specs=[
            pl.BlockSpec((bq, KSEL * CHUNK), lambda i: (i, 0)),
            pl.BlockSpec((bq, KSEL), lambda i: (i, 0)),
        ],
        out_specs=[
            pl.BlockSpec((bq, KSEL), lambda i: (i, 0)),
            pl.BlockSpec((bq, KSEL), lambda i: (i, 0)),
        ],
        out_shape=[
            jax.ShapeDtypeStruct((b, KSEL), jnp.float32),
            jax.ShapeDtypeStruct((b, KSEL), jnp.int32),
        ],
    )(cand, rowid)


def _fusion_body(bq, zq_ref, ctx_ref, wq_ref, bq_ref, wkp_ref, bkp_ref, wv_ref,
                 bv_ref, wo_ref, bo_ref, lng_ref, lnb_ref, wf1_ref, bf1_ref,
                 wf2_ref, bf2_ref, wa1_ref, ba1_ref, wa2_ref, ba2_ref,
                 fc_ref, an_ref):
    zq = zq_ref[...]
    ctx = ctx_ref[...]
    cn = ctx / (jnp.sqrt(jnp.sum(ctx * ctx, axis=1, keepdims=True)) + 1e-8)
    q = jnp.dot(zq, wq_ref[...], preferred_element_type=jnp.float32) + bq_ref[...]
    kk = jnp.dot(cn, wkp_ref[...], preferred_element_type=jnp.float32) + bkp_ref[...]
    vv = jnp.dot(cn, wv_ref[...], preferred_element_type=jnp.float32) + bv_ref[...]
    kr = kk.reshape(bq, KSEL, D_MODEL)
    vr = vv.reshape(bq, KSEL, D_MODEL)
    scale = jnp.float32(1.0 / (DH ** 0.5))
    outs = []
    for h in range(NHEAD):
        sl = slice(h * DH, (h + 1) * DH)
        qh = q[:, sl]
        lh = jnp.sum(qh[:, None, :] * kr[:, :, sl], axis=2) * scale
        lh = lh - jnp.max(lh, axis=1, keepdims=True)
        eh = jnp.exp(lh)
        ah = eh / jnp.sum(eh, axis=1, keepdims=True)
        outs.append(jnp.sum(ah[:, :, None] * vr[:, :, sl], axis=1))
    o = jnp.concatenate(outs, axis=1)
    c_attn = jnp.dot(o, wo_ref[...], preferred_element_type=jnp.float32) + bo_ref[...]
    x = c_attn + zq
    mu = jnp.mean(x, axis=1, keepdims=True)
    xm = x - mu
    var = jnp.mean(xm * xm, axis=1, keepdims=True)
    zf = xm / jnp.sqrt(var + 1e-5) * lng_ref[...] + lnb_ref[...]
    fused = jnp.concatenate([zq, zf], axis=1)
    f1 = jax.nn.gelu(jnp.dot(fused, wf1_ref[...], preferred_element_type=jnp.float32)
                     + bf1_ref[...])
    fc_ref[...] = jnp.dot(f1, wf2_ref[...], preferred_element_type=jnp.float32) + bf2_ref[...]
    a1 = jax.nn.gelu(jnp.dot(fused, wa1_ref[...], preferred_element_type=jnp.float32)
                     + ba1_ref[...])
    a2 = jnp.sum(a1 * wa2_ref[...], axis=1, keepdims=True) + ba2_ref[...]
    an_ref[...] = 1.0 / (1.0 + jnp.exp(-a2))


def _fusion_call(z_q, ctx, wqt, bq1, wkpt, bkp1, wvt, bv1, wot, bo1, lng1, lnb1,
                 wf1t, bf11, wf2t, bf21, wa1t, ba11, wa2, ba21, horizon):
    b = z_q.shape[0]
    bq = 128 if b % 128 == 0 else b
    nblk = b // bq
    body = functools.partial(_fusion_body, bq)
    const = lambda shape: pl.BlockSpec(shape, lambda i: (0, 0))
    return pl.pallas_call(
        body,
        grid=(nblk,),
        in_specs=[
            pl.BlockSpec((bq, D_MODEL), lambda i: (i, 0)),
            pl.BlockSpec((bq * KSEL, D_MODEL), lambda i: (i, 0)),
            const((D_MODEL, D_MODEL)), const((1, D_MODEL)),
            const((D_MODEL, D_MODEL)), const((1, D_MODEL)),
            const((D_MODEL, D_MODEL)), const((1, D_MODEL)),
            const((D_MODEL, D_MODEL)), const((1, D_MODEL)),
            const((1, D_MODEL)), const((1, D_MODEL)),
            const((2 * D_MODEL, D_MODEL)), const((1, D_MODEL)),
            const((D_MODEL, horizon)), const((1, horizon)),
            const((2 * D_MODEL, 64)), const((1, 64)),
            const((1, 64)), const((1, 1)),
        ],
        out_specs=[
            pl.BlockSpec((bq, horizon), lambda i: (i, 0)),
            pl.BlockSpec((bq, 1), lambda i: (i, 0)),
        ],
        out_shape=[
            jax.ShapeDtypeStruct((b, horizon), jnp.float32),
            jax.ShapeDtypeStruct((b, 1), jnp.float32),
        ],
    )(z_q, ctx, wqt, bq1, wkpt, bkp1, wvt, bv1, wot, bo1, lng1, lnb1,
      wf1t, bf11, wf2t, bf21, wa1t, ba11, wa2, ba21)


def kernel(z_q, keys, k, Wq, bq, Wkp, bkp, Wv, bv, Wo, bo, ln_g, ln_b,
           Wf1, bf1, Wf2, bf2, Wa1, ba1, Wa2, ba2):
    del k
    b, d = z_q.shape
    n = keys.shape[0]
    npad = ((n + TILE - 1) // TILE) * TILE
    nchunk = npad // CHUNK
    horizon = Wf2.shape[0]
    qn = z_q / (jnp.linalg.norm(z_q, axis=-1, keepdims=True) + 1e-8)
    knorm = jnp.linalg.norm(keys, axis=-1, keepdims=True) + 1e-8
    knorm_pad = jnp.pad(knorm, ((0, npad - n), (0, 0)), constant_values=1.0)

    sims, cm3 = _sims_call(qn, keys, knorm_pad, npad, n)
    cm = cm3.transpose(1, 0, 2).reshape(b, nchunk)
    rowid = _chunksel_call(cm)
    cand = _sc_gather(sims.reshape(b * nchunk, CHUNK), rowid.reshape(-1), CHUNK)
    top_vals, top_idx = _select_call(cand.reshape(b, KSEL * CHUNK), rowid, nchunk)
    ctx = _sc_gather(keys, top_idx.reshape(-1), d)

    forecast, anomaly = _fusion_call(
        z_q, ctx, Wq.T, bq.reshape(1, -1), Wkp.T, bkp.reshape(1, -1),
        Wv.T, bv.reshape(1, -1), Wo.T, bo.reshape(1, -1),
        ln_g.reshape(1, -1), ln_b.reshape(1, -1), Wf1.T, bf1.reshape(1, -1),
        Wf2.T, bf2.reshape(1, -1), Wa1.T, ba1.reshape(1, -1),
        Wa2, ba2.reshape(1, 1), horizon)
    return forecast, anomaly, top_vals, top_idx


# speedup vs baseline: 1.0238x; 1.0047x over previous
import functools

import jax
import jax.numpy as jnp
import numpy as np
from jax import lax
from jax.experimental import pallas as pl
from jax.experimental.pallas import tpu as pltpu
from jax.experimental.pallas import tpu_sc as plsc

D_MODEL = 512
NHEAD = 8
DH = D_MODEL // NHEAD
KSEL = 16
CHUNK = 128
TILE = 2048
NEG = float(np.float32(-3e38))

SC_NC = 2
SC_NS = 16
SC_NW = SC_NC * SC_NS


def _sims_body(b, n, zq_ref, keys_ref, knorm_ref, sims_ref, cm_ref):
    ti = pl.program_id(0)
    cpt = TILE // CHUNK
    qn = zq_ref[...]
    kn = keys_ref[...] / knorm_ref[...]
    sims = lax.dot_general(qn, kn, (((1,), (1,)), ((), ())),
                           preferred_element_type=jnp.float32)
    col = ti * TILE + lax.broadcasted_iota(jnp.int32, (b, TILE), 1)
    sims = jnp.where(col < n, sims, NEG)
    sims_ref[...] = sims
    cm_ref[0] = jnp.max(sims.reshape(b, cpt, CHUNK), axis=2)


def _sims_call(qn, keys, knorm_pad, npad, n):
    b = qn.shape[0]
    nt = npad // TILE
    cpt = TILE // CHUNK
    body = functools.partial(_sims_body, b, n)
    return pl.pallas_call(
        body,
        grid=(nt,),
        in_specs=[
            pl.BlockSpec((b, D_MODEL), lambda i: (0, 0)),
            pl.BlockSpec((TILE, D_MODEL), lambda i: (i, 0)),
            pl.BlockSpec((TILE, 1), lambda i: (i, 0)),
        ],
        out_specs=[
            pl.BlockSpec((b, TILE), lambda i: (0, i)),
            pl.BlockSpec((1, b, cpt), lambda i: (i, 0, 0)),
        ],
        out_shape=[
            jax.ShapeDtypeStruct((b, npad), jnp.float32),
            jax.ShapeDtypeStruct((nt, b, cpt), jnp.float32),
        ],
    )(qn, keys, knorm_pad)


def _chunksel_body(b, nchunk, cm_ref, rowid_ref):
    work = cm_ref[...]
    gid = lax.broadcasted_iota(jnp.int32, (b, nchunk), 1)
    big = jnp.int32(2 ** 30)
    cols = []
    for _ in range(KSEL):
        m = jnp.max(work, axis=1)
        p = jnp.min(jnp.where(work == m[:, None], gid, big), axis=1)
        cols.append(p.reshape(b, 1))
        work = jnp.where(gid == p[:, None], NEG, work)
    brow = lax.broadcasted_iota(jnp.int32, (b, 1), 0) * nchunk
    rowid_ref[...] = jnp.concatenate(cols, axis=1) + brow


def _chunksel_call(cm):
    b, nchunk = cm.shape
    body = functools.partial(_chunksel_body, b, nchunk)
    return pl.pallas_call(
        body,
        out_shape=jax.ShapeDtypeStruct((b, KSEL), jnp.int32),
    )(cm)


def _sc_gather(table, idx, d):
    rows = idx.shape[0]
    per_w = rows // SC_NW
    rp = 128 if d <= 128 else 64
    j_steps = per_w // rp
    idx3 = idx.reshape(SC_NW, j_steps, rp)
    mesh = plsc.VectorSubcoreMesh(core_axis_name="c", subcore_axis_name="s")

    @functools.partial(
        pl.kernel,
        mesh=mesh,
        out_type=jax.ShapeDtypeStruct((rows, d), jnp.float32),
        scratch_types=[
            pltpu.VMEM((j_steps, rp), jnp.int32),
            pltpu.VMEM((2, rp, d), jnp.float32),
            pltpu.SemaphoreType.DMA,
        ],
    )
    def k(table_hbm, idx_hbm, out_hbm, idx_v, rows_v, sem):
        wid = lax.axis_index("s") * SC_NC + lax.axis_index("c")
        base = wid * per_w
        pltpu.sync_copy(idx_hbm.at[wid], idx_v)
        cps = []
        for j in range(j_steps):
            cps.append(pltpu.async_copy(
                table_hbm.at[idx_v.at[j]], rows_v.at[j % 2], sem))
            if j >= 1:
                cps[j - 1].wait()
                pltpu.sync_copy(rows_v.at[(j - 1) % 2],
                                out_hbm.at[pl.ds(base + (j - 1) * rp, rp)])
        cps[j_steps - 1].wait()
        pltpu.sync_copy(rows_v.at[(j_steps - 1) % 2],
                        out_hbm.at[pl.ds(base + (j_steps - 1) * rp, rp)])

    return k(table, idx3)


def _select_body(b, nchunk, cand_ref, rowid_ref, tv_ref, ti_ref):
    bi = pl.program_id(0)
    cand = cand_ref[...]
    rowid = rowid_ref[...]
    brow = (bi * b + lax.broadcasted_iota(jnp.int32, (b, KSEL), 0)) * nchunk
    base = (rowid - brow) * CHUNK
    lane = lax.broadcasted_iota(jnp.int32, (b, KSEL, CHUNK), 2)
    gidx = (base.reshape(b, KSEL, 1) + lane).reshape(b, KSEL * CHUNK)
    big = jnp.int32(2 ** 30)
    vals, idxs = [], []
    for _ in range(KSEL):
        m = jnp.max(cand, axis=1)
        p = jnp.min(jnp.where(cand == m[:, None], gidx, big), axis=1)
        vals.append(m.reshape(b, 1))
        idxs.append(p.reshape(b, 1))
        cand = jnp.where(gidx == p[:, None], NEG, cand)
    tv_ref[...] = jnp.concatenate(vals, axis=1)
    ti_ref[...] = jnp.concatenate(idxs, axis=1)


def _select_call(cand, rowid, nchunk):
    b = cand.shape[0]
    bq = 256 if b % 256 == 0 else b
    nblk = b // bq
    body = functools.partial(_select_body, bq, nchunk)
    return pl.pallas_call(
        body,
        grid=(nblk,),
        in_specs=[
            pl.BlockSpec((bq, KSEL * CHUNK), lambda i: (i, 0)),
            pl.BlockSpec((bq, KSEL), lambda i: (i, 0)),
        ],
        out_specs=[
            pl.BlockSpec((bq, KSEL), lambda i: (i, 0)),
            pl.BlockSpec((bq, KSEL), lambda i: (i, 0)),
        ],
        out_shape=[
            jax.ShapeDtypeStruct((b, KSEL), jnp.float32),
            jax.ShapeDtypeStruct((b, KSEL), jnp.int32),
        ],
    )(cand, rowid)


def _fusion_body(bq, zq_ref, ctx_ref, wq_ref, bq_ref, wkp_ref, bkp_ref, wv_ref,
                 bv_ref, wo_ref, bo_ref, lng_ref, lnb_ref, wf1_ref, bf1_ref,
                 wf2_ref, bf2_ref, wa1_ref, ba1_ref, wa2_ref, ba2_ref,
                 fc_ref, an_ref):
    zq = zq_ref[...]
    ctx = ctx_ref[...]
    cn = ctx / (jnp.sqrt(jnp.sum(ctx * ctx, axis=1, keepdims=True)) + 1e-8)
    q = jnp.dot(zq, wq_ref[...], preferred_element_type=jnp.float32) + bq_ref[...]
    kk = jnp.dot(cn, wkp_ref[...], preferred_element_type=jnp.float32) + bkp_ref[...]
    vv = jnp.dot(cn, wv_ref[...], preferred_element_type=jnp.float32) + bv_ref[...]
    kr = kk.reshape(bq, KSEL, D_MODEL)
    vr = vv.reshape(bq, KSEL, D_MODEL)
    scale = jnp.float32(1.0 / (DH ** 0.5))
    outs = []
    for h in range(NHEAD):
        sl = slice(h * DH, (h + 1) * DH)
        qh = q[:, sl]
        lh = jnp.sum(qh[:, None, :] * kr[:, :, sl], axis=2) * scale
        lh = lh - jnp.max(lh, axis=1, keepdims=True)
        eh = jnp.exp(lh)
        ah = eh / jnp.sum(eh, axis=1, keepdims=True)
        outs.append(jnp.sum(ah[:, :, None] * vr[:, :, sl], axis=1))
    o = jnp.concatenate(outs, axis=1)
    c_attn = jnp.dot(o, wo_ref[...], preferred_element_type=jnp.float32) + bo_ref[...]
    x = c_attn + zq
    mu = jnp.mean(x, axis=1, keepdims=True)
    xm = x - mu
    var = jnp.mean(xm * xm, axis=1, keepdims=True)
    zf = xm / jnp.sqrt(var + 1e-5) * lng_ref[...] + lnb_ref[...]
    fused = jnp.concatenate([zq, zf], axis=1)
    f1 = jax.nn.gelu(jnp.dot(fused, wf1_ref[...], preferred_element_type=jnp.float32)
                     + bf1_ref[...])
    fc_ref[...] = jnp.dot(f1, wf2_ref[...], preferred_element_type=jnp.float32) + bf2_ref[...]
    a1 = jax.nn.gelu(jnp.dot(fused, wa1_ref[...], preferred_element_type=jnp.float32)
                     + ba1_ref[...])
    a2 = jnp.sum(a1 * wa2_ref[...], axis=1, keepdims=True) + ba2_ref[...]
    an_ref[...] = 1.0 / (1.0 + jnp.exp(-a2))


def _fusion_call(z_q, ctx, wqt, bq1, wkpt, bkp1, wvt, bv1, wot, bo1, lng1, lnb1,
                 wf1t, bf11, wf2t, bf21, wa1t, ba11, wa2, ba21, horizon):
    b = z_q.shape[0]
    bq = 256 if b % 256 == 0 else b
    nblk = b // bq
    body = functools.partial(_fusion_body, bq)
    const = lambda shape: pl.BlockSpec(shape, lambda i: (0, 0))
    return pl.pallas_call(
        body,
        grid=(nblk,),
        in_specs=[
            pl.BlockSpec((bq, D_MODEL), lambda i: (i, 0)),
            pl.BlockSpec((bq * KSEL, D_MODEL), lambda i: (i, 0)),
            const((D_MODEL, D_MODEL)), const((1, D_MODEL)),
            const((D_MODEL, D_MODEL)), const((1, D_MODEL)),
            const((D_MODEL, D_MODEL)), const((1, D_MODEL)),
            const((D_MODEL, D_MODEL)), const((1, D_MODEL)),
            const((1, D_MODEL)), const((1, D_MODEL)),
            const((2 * D_MODEL, D_MODEL)), const((1, D_MODEL)),
            const((D_MODEL, horizon)), const((1, horizon)),
            const((2 * D_MODEL, 64)), const((1, 64)),
            const((1, 64)), const((1, 1)),
        ],
        out_specs=[
            pl.BlockSpec((bq, horizon), lambda i: (i, 0)),
            pl.BlockSpec((bq, 1), lambda i: (i, 0)),
        ],
        out_shape=[
            jax.ShapeDtypeStruct((b, horizon), jnp.float32),
            jax.ShapeDtypeStruct((b, 1), jnp.float32),
        ],
    )(z_q, ctx, wqt, bq1, wkpt, bkp1, wvt, bv1, wot, bo1, lng1, lnb1,
      wf1t, bf11, wf2t, bf21, wa1t, ba11, wa2, ba21)


def kernel(z_q, keys, k, Wq, bq, Wkp, bkp, Wv, bv, Wo, bo, ln_g, ln_b,
           Wf1, bf1, Wf2, bf2, Wa1, ba1, Wa2, ba2):
    del k
    b, d = z_q.shape
    n = keys.shape[0]
    npad = ((n + TILE - 1) // TILE) * TILE
    nchunk = npad // CHUNK
    horizon = Wf2.shape[0]
    qn = z_q / (jnp.linalg.norm(z_q, axis=-1, keepdims=True) + 1e-8)
    knorm = jnp.linalg.norm(keys, axis=-1, keepdims=True) + 1e-8
    knorm_pad = jnp.pad(knorm, ((0, npad - n), (0, 0)), constant_values=1.0)

    sims, cm3 = _sims_call(qn, keys, knorm_pad, npad, n)
    cm = cm3.transpose(1, 0, 2).reshape(b, nchunk)
    rowid = _chunksel_call(cm)
    cand = _sc_gather(sims.reshape(b * nchunk, CHUNK), rowid.reshape(-1), CHUNK)
    top_vals, top_idx = _select_call(cand.reshape(b, KSEL * CHUNK), rowid, nchunk)
    ctx = _sc_gather(keys, top_idx.reshape(-1), d)

    forecast, anomaly = _fusion_call(
        z_q, ctx, Wq.T, bq.reshape(1, -1), Wkp.T, bkp.reshape(1, -1),
        Wv.T, bv.reshape(1, -1), Wo.T, bo.reshape(1, -1),
        ln_g.reshape(1, -1), ln_b.reshape(1, -1), Wf1.T, bf1.reshape(1, -1),
        Wf2.T, bf2.reshape(1, -1), Wa1.T, ba1.reshape(1, -1),
        Wa2, ba2.reshape(1, 1), horizon)
    return forecast, anomaly, top_vals, top_idx
